# baseline (speedup 1.0000x reference)
import functools

import jax
import jax.numpy as jnp
from jax import lax
from jax.experimental import pallas as pl
from jax.experimental.pallas import tpu as pltpu
import jax.experimental.pallas.tpu_sc as plsc

N_NODES = 10000
NPAD = 10112
N_TILES = 16
K_CH = 128


def _round_up(v, m):
    return ((v + m - 1) // m) * m


def _sweep12(C, Cd, H, epad, npad, k):
    mesh = plsc.VectorSubcoreMesh(core_axis_name="c", subcore_axis_name="s", num_cores=2, num_subcores=16)
    per_tile = epad // N_TILES
    n_chunks = per_tile // k
    rpt = npad // N_TILES
    groups = k // 16

    n_pairs = n_chunks // 2

    def body(srcd, dstd, as_h, ad_h, h_h, zc, zh, acc_o, den_o,
             src_v, dst_v, asr, adr, hrows, contrib, exb, acc_s, den_s, sem):
        c = lax.axis_index("c")
        s = lax.axis_index("s")
        iota16 = lax.iota(jnp.int32, 16)
        r0 = s * rpt
        pltpu.sync_copy(zc.at[pl.ds(r0, rpt)], acc_s.at[pl.ds(r0, rpt)])
        pltpu.sync_copy(zh.at[pl.ds(r0, rpt)], den_s.at[pl.ds(r0, rpt)])
        pltpu.sync_copy(zh.at[pl.ds(0, k)], exb.at[0])
        pltpu.sync_copy(zh.at[pl.ds(0, k)], exb.at[1])
        plsc.subcore_barrier()
        base0 = s * per_tile

        def issue(ci, b):
            base = base0 + ci * k
            pltpu.sync_copy(srcd.at[pl.ds(base, k)], src_v.at[b])
            pltpu.sync_copy(dstd.at[pl.ds(base, k)], dst_v.at[b])
            pltpu.async_copy(as_h.at[src_v.at[b]], asr.at[b], sem)
            pltpu.async_copy(ad_h.at[dst_v.at[b]], adr.at[b], sem)
            pltpu.async_copy(h_h.at[c].at[src_v.at[b]], hrows.at[b], sem)

        def wait_gathers(b):
            pltpu.make_async_copy(as_h.at[src_v.at[b]], asr.at[b], sem).wait()
            pltpu.make_async_copy(ad_h.at[dst_v.at[b]], adr.at[b], sem).wait()
            pltpu.make_async_copy(
                h_h.at[c].at[src_v.at[b]], hrows.at[b], sem).wait()

        def compute(b):
            def group(g, carry2):
                k16 = g * 16 + iota16
                for lh in range(H):
                    gcol = jnp.full((16,), c * H + lh, jnp.int32)
                    a1 = plsc.load_gather(asr.at[b], [k16, gcol])
                    a2 = plsc.load_gather(adr.at[b], [k16, gcol])
                    e = a1 + a2
                    ex = jnp.exp(jnp.where(e > 0, e, 0.2 * e))
                    plsc.store_scatter(
                        exb.at[b], [k16, jnp.full((16,), lh, jnp.int32)], ex)
                    for cd in range(Cd):
                        col16 = jnp.full((16,), lh * Cd + cd, jnp.int32)
                        hv = plsc.load_gather(hrows.at[b], [k16, col16])
                        plsc.store_scatter(contrib.at[b], [k16, col16], ex * hv)
                return carry2

            lax.fori_loop(0, groups, group, 0)

        issue(0, 0)

        def pair(i, carry):
            for b in (0, 1):
                wait_gathers(b)
                if b == 0:
                    issue(2 * i + 1, 1)
                else:
                    @pl.when(i < n_pairs - 1)
                    def _():
                        issue(2 * i + 2, 0)
                compute(b)
                pltpu.sync_copy(contrib.at[b], acc_s.at[dst_v.at[b]], add=True)
                pltpu.sync_copy(exb.at[b], den_s.at[dst_v.at[b]], add=True)
            return carry

        lax.fori_loop(0, n_pairs, pair, 0)
        plsc.subcore_barrier()
        pltpu.sync_copy(acc_s.at[pl.ds(r0, rpt)], acc_o.at[c].at[pl.ds(r0, rpt)])
        pltpu.sync_copy(den_s.at[pl.ds(r0, rpt)], den_o.at[c].at[pl.ds(r0, rpt)])

    return pl.kernel(
        body,
        out_type=(
            jax.ShapeDtypeStruct((2, npad, C), jnp.float32),
            jax.ShapeDtypeStruct((2, npad, 8), jnp.float32),
        ),
        mesh=mesh,
        compiler_params=pltpu.CompilerParams(needs_layout_passes=False, use_tc_tiling_on_sc=False),
        scratch_types=[
            pltpu.VMEM((2, k), jnp.int32),
            pltpu.VMEM((2, k), jnp.int32),
            pltpu.VMEM((2, k, 2 * H), jnp.float32),
            pltpu.VMEM((2, k, 2 * H), jnp.float32),
            pltpu.VMEM((2, k, C), jnp.float32),
            pltpu.VMEM((2, k, C), jnp.float32),
            pltpu.VMEM((2, k, 8), jnp.float32),
            pltpu.VMEM_SHARED((npad, C), jnp.float32),
            pltpu.VMEM_SHARED((npad, 8), jnp.float32),
            pltpu.SemaphoreType.DMA,
        ],
    )


def _sweep3(epad, npad, k):
    mesh = plsc.VectorSubcoreMesh(core_axis_name="c", subcore_axis_name="s", num_cores=2, num_subcores=16)
    per_core = epad // 2
    per_tile = per_core // N_TILES
    n_chunks = per_tile // k
    rpt = npad // N_TILES
    groups = k // 16

    def body(srcd, dstd, as_h, ad_h, h_h, zc, acc_o,
             src_v, dst_v, asr, adr, hrows, contrib, acc_s, sem):
        c = lax.axis_index("c")
        s = lax.axis_index("s")
        iota16 = lax.iota(jnp.int32, 16)
        r0 = s * rpt
        pltpu.sync_copy(zc.at[pl.ds(r0, rpt)], acc_s.at[pl.ds(r0, rpt)])
        pltpu.sync_copy(zc.at[pl.ds(0, k)], contrib.at[0])
        pltpu.sync_copy(zc.at[pl.ds(0, k)], contrib.at[1])
        plsc.subcore_barrier()
        base0 = c * per_core + s * per_tile

        def issue(ci, b):
            base = base0 + ci * k
            pltpu.sync_copy(srcd.at[pl.ds(base, k)], src_v.at[b])
            pltpu.sync_copy(dstd.at[pl.ds(base, k)], dst_v.at[b])
            pltpu.async_copy(as_h.at[src_v.at[b]], asr.at[b], sem)
            pltpu.async_copy(ad_h.at[dst_v.at[b]], adr.at[b], sem)
            pltpu.async_copy(h_h.at[src_v.at[b]], hrows.at[b], sem)

        def wait_gathers(b):
            pltpu.make_async_copy(as_h.at[src_v.at[b]], asr.at[b], sem).wait()
            pltpu.make_async_copy(ad_h.at[dst_v.at[b]], adr.at[b], sem).wait()
            pltpu.make_async_copy(h_h.at[src_v.at[b]], hrows.at[b], sem).wait()

        def compute(b):
            def group(g, carry2):
                k16 = g * 16 + iota16
                a1 = plsc.load_gather(asr.at[b], [k16])
                a2 = plsc.load_gather(adr.at[b], [k16])
                e = a1 + a2
                ex = jnp.exp(jnp.where(e > 0, e, 0.2 * e))
                for cd in range(16):
                    col16 = jnp.full((16,), cd, jnp.int32)
                    hv = plsc.load_gather(hrows.at[b], [k16, col16])
                    plsc.store_scatter(contrib.at[b], [k16, col16], ex * hv)
                plsc.store_scatter(
                    contrib.at[b], [k16, jnp.full((16,), 16, jnp.int32)], ex)
                return carry2

            lax.fori_loop(0, groups, group, 0)

        n_pairs = n_chunks // 2
        issue(0, 0)

        def pair(i, carry):
            for b in (0, 1):
                wait_gathers(b)
                if b == 0:
                    issue(2 * i + 1, 1)
                else:
                    @pl.when(i < n_pairs - 1)
                    def _():
                        issue(2 * i + 2, 0)
                compute(b)
                pltpu.sync_copy(contrib.at[b], acc_s.at[dst_v.at[b]], add=True)
            return carry

        lax.fori_loop(0, n_pairs, pair, 0)
        plsc.subcore_barrier()
        pltpu.sync_copy(acc_s.at[pl.ds(r0, rpt)], acc_o.at[c].at[pl.ds(r0, rpt)])

    return pl.kernel(
        body,
        out_type=jax.ShapeDtypeStruct((2, npad, 32), jnp.float32),
        mesh=mesh,
        compiler_params=pltpu.CompilerParams(needs_layout_passes=False, use_tc_tiling_on_sc=False),
        scratch_types=[
            pltpu.VMEM((2, k), jnp.int32),
            pltpu.VMEM((2, k), jnp.int32),
            pltpu.VMEM((2, k), jnp.float32),
            pltpu.VMEM((2, k), jnp.float32),
            pltpu.VMEM((2, k, 16), jnp.float32),
            pltpu.VMEM((2, k, 32), jnp.float32),
            pltpu.VMEM_SHARED((npad, 32), jnp.float32),
            pltpu.SemaphoreType.DMA,
        ],
    )


def _tc_first(x, W, As, Ad, C):
    n = x.shape[0]
    heads = As.shape[1]

    def body(x_ref, w_ref, as_ref, ad_ref, h_out, s_out, d_out):
        h = jnp.dot(x_ref[...], w_ref[...], preferred_element_type=jnp.float32)
        s_out[...] = jnp.dot(h, as_ref[...], preferred_element_type=jnp.float32)
        d_out[...] = jnp.dot(h, ad_ref[...], preferred_element_type=jnp.float32)
        h_out[0] = h[:, :C]
        h_out[1] = h[:, C:]

    return pl.pallas_call(
        body,
        out_shape=(
            jax.ShapeDtypeStruct((2, n, C), jnp.float32),
            jax.ShapeDtypeStruct((n, heads), jnp.float32),
            jax.ShapeDtypeStruct((n, heads), jnp.float32),
        ),
    )(x, W, As, Ad)


def _tc_norm(acc, den, b, g, be, Cd, H):
    n = acc.shape[1]

    def body(acc_ref, den_ref, b_ref, g_ref, be_ref, x_out):
        parts = []
        for c in range(2):
            a = acc_ref[c]
            d = den_ref[c]
            denr = jnp.concatenate(
                [jnp.broadcast_to(d[:, h:h + 1], (n, Cd)) for h in range(H)],
                axis=1)
            parts.append(a / (denr + 1e-16))
        o = jnp.concatenate(parts, axis=1) + b_ref[...]
        mu = jnp.mean(o, axis=0)
        var = jnp.mean((o - mu) ** 2, axis=0)
        xb = (o - mu) * lax.rsqrt(var + 1e-5) * g_ref[...] + be_ref[...]
        x_out[...] = jnp.where(xb > 0, xb, jnp.exp(xb) - 1.0)

    return pl.pallas_call(
        body,
        out_shape=jax.ShapeDtypeStruct((n, 2 * acc.shape[2]), jnp.float32),
        compiler_params=pltpu.CompilerParams(
            vmem_limit_bytes=64 * 1024 * 1024),
    )(acc, den, b, g, be)


def _tc_mid(acc, den, b, g, be, W, As, Ad, Cd, H, split_out):
    xa = _tc_norm(acc, den, b, g, be, Cd, H)
    n = xa.shape[0]
    Fout = W.shape[1]
    heads = As.shape[1]

    def body(x_ref, w_ref, as_ref, ad_ref, h_out, s_out, d_out):
        h = jnp.dot(x_ref[...], w_ref[...], preferred_element_type=jnp.float32)
        s_out[...] = jnp.dot(h, as_ref[...], preferred_element_type=jnp.float32)
        d_out[...] = jnp.dot(h, ad_ref[...], preferred_element_type=jnp.float32)
        if split_out:
            h_out[0] = h[:, :Fout // 2]
            h_out[1] = h[:, Fout // 2:]
        else:
            h_out[...] = h

    h_shape = ((2, n, Fout // 2) if split_out else (n, Fout))
    return pl.pallas_call(
        body,
        out_shape=(
            jax.ShapeDtypeStruct(h_shape, jnp.float32),
            jax.ShapeDtypeStruct((n, heads), jnp.float32),
            jax.ShapeDtypeStruct((n, heads), jnp.float32),
        ),
    )(xa, W, As, Ad)


def _tc_final(acc, b):
    n = acc.shape[1]

    def body(acc_ref, b_ref, out_ref):
        a = acc_ref[0] + acc_ref[1]
        h = a[:, :16] / (a[:, 16:17] + 1e-16) + b_ref[...]
        m = jnp.max(h, axis=1, keepdims=True)
        lse = m + jnp.log(jnp.sum(jnp.exp(h - m), axis=1, keepdims=True))
        out_ref[...] = h - lse

    return pl.pallas_call(
        body,
        out_shape=jax.ShapeDtypeStruct((n, 16), jnp.float32),
    )(acc, b)


def _block_diag_proj(a):
    heads, Cd = a.shape
    A = a[:, :, None] * jnp.eye(heads, dtype=a.dtype)[:, None, :]
    return A.reshape(heads * Cd, heads)


def _pad_rows(a, npad):
    pad = [(0, npad - a.shape[0])] + [(0, 0)] * (a.ndim - 1)
    return jnp.pad(a, pad)


def kernel(x, edge_index, W1, a1s, a1d, b1, g1, be1,
           W2, a2s, a2d, b2, g2, be2, W3, a3s, a3d, b3):
    n, _ = x.shape
    e = edge_index.shape[1]
    e_tot = e + n
    epad = _round_up(e_tot, 64 * K_CH)
    npad = NPAD

    loops = jnp.arange(n, dtype=jnp.int32)
    srcd = jnp.concatenate(
        [edge_index[0].astype(jnp.int32), loops,
         jnp.zeros((epad - e_tot,), jnp.int32)])
    dstd = jnp.concatenate(
        [edge_index[1].astype(jnp.int32), loops,
         jnp.full((epad - e_tot,), n, jnp.int32)])

    A1s, A1d = _block_diag_proj(a1s), _block_diag_proj(a1d)
    A2s, A2d = _block_diag_proj(a2s), _block_diag_proj(a2d)
    A3s, A3d = _block_diag_proj(a3s), _block_diag_proj(a3d)

    zc64 = jnp.zeros((npad, 64), jnp.float32)
    zc128 = jnp.zeros((npad, 128), jnp.float32)
    zc32 = jnp.zeros((npad, 32), jnp.float32)
    zh8 = jnp.zeros((npad, 8), jnp.float32)

    h1, as1, ad1 = _tc_first(x, W1, A1s, A1d, 64)
    acc1, den1 = _sweep12(64, 16, 4, epad, npad, 64)(
        srcd, dstd, _pad_rows(as1, npad), _pad_rows(ad1, npad),
        h1, zc64, zh8)

    h2, as2, ad2 = _tc_mid(acc1[:, :n], den1[:, :n], b1, g1, be1,
                           W2, A2s, A2d, 16, 4, True)
    acc2, den2 = _sweep12(128, 32, 4, epad, npad, 64)(
        srcd, dstd, _pad_rows(as2, npad), _pad_rows(ad2, npad),
        h2, zc128, zh8)

    h3, as3, ad3 = _tc_mid(acc2[:, :n], den2[:, :n], b2, g2, be2,
                           W3, A3s, A3d, 32, 4, False)
    acc3 = _sweep3(epad, npad, K_CH)(
        srcd, dstd, _pad_rows(as3[:, 0], npad), _pad_rows(ad3[:, 0], npad),
        h3, zc32)

    return _tc_final(acc3[:, :n], b3)


# speedup vs baseline: 3.4788x; 3.4788x over previous
import functools

import jax
import jax.numpy as jnp
from jax import lax
from jax.experimental import pallas as pl
from jax.experimental.pallas import tpu as pltpu
import jax.experimental.pallas.tpu_sc as plsc

N_NODES = 10000
NPAD = 10112
N_TILES = 16
K_CH = 128


def _round_up(v, m):
    return ((v + m - 1) // m) * m


def _sweep12(C, Cd, H, epad, npad, k):
    mesh = plsc.VectorSubcoreMesh(core_axis_name="c", subcore_axis_name="s", num_cores=2, num_subcores=16)
    per_tile = epad // N_TILES
    n_chunks = per_tile // k
    rpt = npad // N_TILES
    groups = k // 16

    n_pairs = n_chunks // 2

    def body(srcd, dstd, as_h, ad_h, h_h, zc, zh, acc_o, den_o,
             src_v, dst_v, asr, adr, hrows, exb, acc_s, den_s, sem):
        c = lax.axis_index("c")
        s = lax.axis_index("s")
        iota16 = lax.iota(jnp.int32, 16)
        r0 = s * rpt
        pltpu.sync_copy(zc.at[pl.ds(r0, rpt)], acc_s.at[pl.ds(r0, rpt)])
        pltpu.sync_copy(zh.at[pl.ds(r0, rpt)], den_s.at[pl.ds(r0, rpt)])
        pltpu.sync_copy(zh.at[pl.ds(0, k)], exb.at[0])
        pltpu.sync_copy(zh.at[pl.ds(0, k)], exb.at[1])
        plsc.subcore_barrier()
        base0 = s * per_tile

        def issue(ci, b):
            base = base0 + ci * k
            pltpu.sync_copy(srcd.at[pl.ds(base, k)], src_v.at[b])
            pltpu.sync_copy(dstd.at[pl.ds(base, k)], dst_v.at[b])
            pltpu.async_copy(as_h.at[src_v.at[b]], asr.at[b], sem)
            pltpu.async_copy(ad_h.at[dst_v.at[b]], adr.at[b], sem)
            pltpu.async_copy(h_h.at[c].at[src_v.at[b]], hrows.at[b], sem)

        def wait_gathers(b):
            pltpu.make_async_copy(as_h.at[src_v.at[b]], asr.at[b], sem).wait()
            pltpu.make_async_copy(ad_h.at[dst_v.at[b]], adr.at[b], sem).wait()
            pltpu.make_async_copy(
                h_h.at[c].at[src_v.at[b]], hrows.at[b], sem).wait()

        def compute(b):
            def group(g, carry2):
                k16 = g * 16 + iota16
                for lh in range(H):
                    gcol = jnp.full((16,), c * H + lh, jnp.int32)
                    a1 = plsc.load_gather(asr.at[b], [k16, gcol])
                    a2 = plsc.load_gather(adr.at[b], [k16, gcol])
                    e = a1 + a2
                    ex = jnp.exp(jnp.where(e > 0, e, 0.2 * e))
                    plsc.store_scatter(
                        exb.at[b], [k16, jnp.full((16,), lh, jnp.int32)], ex)
                    for j in range(16):
                        exv = jnp.full((16,), ex[j], jnp.float32)
                        kk = g * 16 + j
                        for t in range(Cd // 16):
                            col = lh * Cd + t * 16
                            hv = hrows.at[b][kk, pl.ds(col, 16)]
                            hrows.at[b][kk, pl.ds(col, 16)] = exv * hv
                return carry2

            lax.fori_loop(0, groups, group, 0)

        issue(0, 0)

        def pair(i, carry):
            for b in (0, 1):
                wait_gathers(b)
                if b == 0:
                    issue(2 * i + 1, 1)
                else:
                    @pl.when(i < n_pairs - 1)
                    def _():
                        issue(2 * i + 2, 0)
                compute(b)
                pltpu.sync_copy(hrows.at[b], acc_s.at[dst_v.at[b]], add=True)
                pltpu.sync_copy(exb.at[b], den_s.at[dst_v.at[b]], add=True)
            return carry

        lax.fori_loop(0, n_pairs, pair, 0)
        plsc.subcore_barrier()
        pltpu.sync_copy(acc_s.at[pl.ds(r0, rpt)], acc_o.at[c].at[pl.ds(r0, rpt)])
        pltpu.sync_copy(den_s.at[pl.ds(r0, rpt)], den_o.at[c].at[pl.ds(r0, rpt)])

    return pl.kernel(
        body,
        out_type=(
            jax.ShapeDtypeStruct((2, npad, C), jnp.float32),
            jax.ShapeDtypeStruct((2, npad, 8), jnp.float32),
        ),
        mesh=mesh,
        compiler_params=pltpu.CompilerParams(needs_layout_passes=False, use_tc_tiling_on_sc=False),
        scratch_types=[
            pltpu.VMEM((2, k), jnp.int32),
            pltpu.VMEM((2, k), jnp.int32),
            pltpu.VMEM((2, k, 2 * H), jnp.float32),
            pltpu.VMEM((2, k, 2 * H), jnp.float32),
            pltpu.VMEM((2, k, C), jnp.float32),
            pltpu.VMEM((2, k, 8), jnp.float32),
            pltpu.VMEM_SHARED((npad, C), jnp.float32),
            pltpu.VMEM_SHARED((npad, 8), jnp.float32),
            pltpu.SemaphoreType.DMA,
        ],
    )


def _sweep3(epad, npad, k):
    mesh = plsc.VectorSubcoreMesh(core_axis_name="c", subcore_axis_name="s", num_cores=2, num_subcores=16)
    per_core = epad // 2
    per_tile = per_core // N_TILES
    n_chunks = per_tile // k
    rpt = npad // N_TILES
    groups = k // 16

    def body(srcd, dstd, as_h, ad_h, h_h, zc, acc_o,
             src_v, dst_v, asr, adr, hrows, contrib, acc_s, sem):
        c = lax.axis_index("c")
        s = lax.axis_index("s")
        iota16 = lax.iota(jnp.int32, 16)
        r0 = s * rpt
        pltpu.sync_copy(zc.at[pl.ds(r0, rpt)], acc_s.at[pl.ds(r0, rpt)])
        pltpu.sync_copy(zc.at[pl.ds(0, k)], contrib.at[0])
        pltpu.sync_copy(zc.at[pl.ds(0, k)], contrib.at[1])
        plsc.subcore_barrier()
        base0 = c * per_core + s * per_tile

        def issue(ci, b):
            base = base0 + ci * k
            pltpu.sync_copy(srcd.at[pl.ds(base, k)], src_v.at[b])
            pltpu.sync_copy(dstd.at[pl.ds(base, k)], dst_v.at[b])
            pltpu.async_copy(as_h.at[src_v.at[b]], asr.at[b], sem)
            pltpu.async_copy(ad_h.at[dst_v.at[b]], adr.at[b], sem)
            pltpu.async_copy(h_h.at[src_v.at[b]], hrows.at[b], sem)

        def wait_gathers(b):
            pltpu.make_async_copy(as_h.at[src_v.at[b]], asr.at[b], sem).wait()
            pltpu.make_async_copy(ad_h.at[dst_v.at[b]], adr.at[b], sem).wait()
            pltpu.make_async_copy(h_h.at[src_v.at[b]], hrows.at[b], sem).wait()

        def compute(b):
            def group(g, carry2):
                k16 = g * 16 + iota16
                a1 = plsc.load_gather(asr.at[b], [k16])
                a2 = plsc.load_gather(adr.at[b], [k16])
                e = a1 + a2
                ex = jnp.exp(jnp.where(e > 0, e, 0.2 * e))
                for cd in range(16):
                    col16 = jnp.full((16,), cd, jnp.int32)
                    hv = plsc.load_gather(hrows.at[b], [k16, col16])
                    plsc.store_scatter(contrib.at[b], [k16, col16], ex * hv)
                plsc.store_scatter(
                    contrib.at[b], [k16, jnp.full((16,), 16, jnp.int32)], ex)
                return carry2

            lax.fori_loop(0, groups, group, 0)

        n_pairs = n_chunks // 2
        issue(0, 0)

        def pair(i, carry):
            for b in (0, 1):
                wait_gathers(b)
                if b == 0:
                    issue(2 * i + 1, 1)
                else:
                    @pl.when(i < n_pairs - 1)
                    def _():
                        issue(2 * i + 2, 0)
                compute(b)
                pltpu.sync_copy(contrib.at[b], acc_s.at[dst_v.at[b]], add=True)
            return carry

        lax.fori_loop(0, n_pairs, pair, 0)
        plsc.subcore_barrier()
        pltpu.sync_copy(acc_s.at[pl.ds(r0, rpt)], acc_o.at[c].at[pl.ds(r0, rpt)])

    return pl.kernel(
        body,
        out_type=jax.ShapeDtypeStruct((2, npad, 32), jnp.float32),
        mesh=mesh,
        compiler_params=pltpu.CompilerParams(needs_layout_passes=False, use_tc_tiling_on_sc=False),
        scratch_types=[
            pltpu.VMEM((2, k), jnp.int32),
            pltpu.VMEM((2, k), jnp.int32),
            pltpu.VMEM((2, k), jnp.float32),
            pltpu.VMEM((2, k), jnp.float32),
            pltpu.VMEM((2, k, 16), jnp.float32),
            pltpu.VMEM((2, k, 32), jnp.float32),
            pltpu.VMEM_SHARED((npad, 32), jnp.float32),
            pltpu.SemaphoreType.DMA,
        ],
    )


def _tc_first(x, W, As, Ad, C):
    n = x.shape[0]
    heads = As.shape[1]

    def body(x_ref, w_ref, as_ref, ad_ref, h_out, s_out, d_out):
        h = jnp.dot(x_ref[...], w_ref[...], preferred_element_type=jnp.float32)
        s_out[...] = jnp.dot(h, as_ref[...], preferred_element_type=jnp.float32)
        d_out[...] = jnp.dot(h, ad_ref[...], preferred_element_type=jnp.float32)
        h_out[0] = h[:, :C]
        h_out[1] = h[:, C:]

    return pl.pallas_call(
        body,
        out_shape=(
            jax.ShapeDtypeStruct((2, n, C), jnp.float32),
            jax.ShapeDtypeStruct((n, heads), jnp.float32),
            jax.ShapeDtypeStruct((n, heads), jnp.float32),
        ),
    )(x, W, As, Ad)


def _tc_norm(acc, den, b, g, be, Cd, H):
    n = acc.shape[1]

    def body(acc_ref, den_ref, b_ref, g_ref, be_ref, x_out):
        parts = []
        for c in range(2):
            a = acc_ref[c]
            d = den_ref[c]
            denr = jnp.concatenate(
                [jnp.broadcast_to(d[:, h:h + 1], (n, Cd)) for h in range(H)],
                axis=1)
            parts.append(a / (denr + 1e-16))
        o = jnp.concatenate(parts, axis=1) + b_ref[...]
        mu = jnp.mean(o, axis=0)
        var = jnp.mean((o - mu) ** 2, axis=0)
        xb = (o - mu) * lax.rsqrt(var + 1e-5) * g_ref[...] + be_ref[...]
        x_out[...] = jnp.where(xb > 0, xb, jnp.exp(xb) - 1.0)

    return pl.pallas_call(
        body,
        out_shape=jax.ShapeDtypeStruct((n, 2 * acc.shape[2]), jnp.float32),
        compiler_params=pltpu.CompilerParams(
            vmem_limit_bytes=64 * 1024 * 1024),
    )(acc, den, b, g, be)


def _tc_mid(acc, den, b, g, be, W, As, Ad, Cd, H, split_out):
    xa = _tc_norm(acc, den, b, g, be, Cd, H)
    n = xa.shape[0]
    Fout = W.shape[1]
    heads = As.shape[1]

    def body(x_ref, w_ref, as_ref, ad_ref, h_out, s_out, d_out):
        h = jnp.dot(x_ref[...], w_ref[...], preferred_element_type=jnp.float32)
        s_out[...] = jnp.dot(h, as_ref[...], preferred_element_type=jnp.float32)
        d_out[...] = jnp.dot(h, ad_ref[...], preferred_element_type=jnp.float32)
        if split_out:
            h_out[0] = h[:, :Fout // 2]
            h_out[1] = h[:, Fout // 2:]
        else:
            h_out[...] = h

    h_shape = ((2, n, Fout // 2) if split_out else (n, Fout))
    return pl.pallas_call(
        body,
        out_shape=(
            jax.ShapeDtypeStruct(h_shape, jnp.float32),
            jax.ShapeDtypeStruct((n, heads), jnp.float32),
            jax.ShapeDtypeStruct((n, heads), jnp.float32),
        ),
    )(xa, W, As, Ad)


def _tc_final(acc, b):
    n = acc.shape[1]

    def body(acc_ref, b_ref, out_ref):
        a = acc_ref[0] + acc_ref[1]
        h = a[:, :16] / (a[:, 16:17] + 1e-16) + b_ref[...]
        m = jnp.max(h, axis=1, keepdims=True)
        lse = m + jnp.log(jnp.sum(jnp.exp(h - m), axis=1, keepdims=True))
        out_ref[...] = h - lse

    return pl.pallas_call(
        body,
        out_shape=jax.ShapeDtypeStruct((n, 16), jnp.float32),
    )(acc, b)


def _block_diag_proj(a):
    heads, Cd = a.shape
    A = a[:, :, None] * jnp.eye(heads, dtype=a.dtype)[:, None, :]
    return A.reshape(heads * Cd, heads)


def _pad_rows(a, npad):
    pad = [(0, npad - a.shape[0])] + [(0, 0)] * (a.ndim - 1)
    return jnp.pad(a, pad)


def kernel(x, edge_index, W1, a1s, a1d, b1, g1, be1,
           W2, a2s, a2d, b2, g2, be2, W3, a3s, a3d, b3):
    n, _ = x.shape
    e = edge_index.shape[1]
    e_tot = e + n
    epad = _round_up(e_tot, 64 * K_CH)
    npad = NPAD

    loops = jnp.arange(n, dtype=jnp.int32)
    srcd = jnp.concatenate(
        [edge_index[0].astype(jnp.int32), loops,
         jnp.zeros((epad - e_tot,), jnp.int32)])
    dstd = jnp.concatenate(
        [edge_index[1].astype(jnp.int32), loops,
         jnp.full((epad - e_tot,), n, jnp.int32)])

    A1s, A1d = _block_diag_proj(a1s), _block_diag_proj(a1d)
    A2s, A2d = _block_diag_proj(a2s), _block_diag_proj(a2d)
    A3s, A3d = _block_diag_proj(a3s), _block_diag_proj(a3d)

    zc64 = jnp.zeros((npad, 64), jnp.float32)
    zc128 = jnp.zeros((npad, 128), jnp.float32)
    zc32 = jnp.zeros((npad, 32), jnp.float32)
    zh8 = jnp.zeros((npad, 8), jnp.float32)

    h1, as1, ad1 = _tc_first(x, W1, A1s, A1d, 64)
    acc1, den1 = _sweep12(64, 16, 4, epad, npad, K_CH)(
        srcd, dstd, _pad_rows(as1, npad), _pad_rows(ad1, npad),
        h1, zc64, zh8)

    h2, as2, ad2 = _tc_mid(acc1[:, :n], den1[:, :n], b1, g1, be1,
                           W2, A2s, A2d, 16, 4, True)
    acc2, den2 = _sweep12(128, 32, 4, epad, npad, K_CH)(
        srcd, dstd, _pad_rows(as2, npad), _pad_rows(ad2, npad),
        h2, zc128, zh8)

    h3, as3, ad3 = _tc_mid(acc2[:, :n], den2[:, :n], b2, g2, be2,
                           W3, A3s, A3d, 32, 4, False)
    acc3 = _sweep3(epad, npad, K_CH)(
        srcd, dstd, _pad_rows(as3[:, 0], npad), _pad_rows(ad3[:, 0], npad),
        h3, zc32)

    return _tc_final(acc3[:, :n], b3)


# speedup vs baseline: 3.6171x; 1.0397x over previous
import functools

import jax
import jax.numpy as jnp
from jax import lax
from jax.experimental import pallas as pl
from jax.experimental.pallas import tpu as pltpu
import jax.experimental.pallas.tpu_sc as plsc

N_NODES = 10000
NPAD = 10112
N_TILES = 16
K_CH = 128


def _round_up(v, m):
    return ((v + m - 1) // m) * m


def _sweep12(C, Cd, H, epad, npad, k):
    mesh = plsc.VectorSubcoreMesh(core_axis_name="c", subcore_axis_name="s", num_cores=2, num_subcores=16)
    per_tile = epad // N_TILES
    n_chunks = per_tile // k
    rpt = npad // N_TILES
    groups = k // 16

    n_pairs = n_chunks // 2

    def body(srcd, dstd, as_h, ad_h, h_h, zc, zh, acc_o, den_o,
             src_v, dst_v, asr, adr, hrows, exb, acc_s, den_s, sem):
        c = lax.axis_index("c")
        s = lax.axis_index("s")
        iota16 = lax.iota(jnp.int32, 16)
        r0 = s * rpt
        pltpu.sync_copy(zc.at[pl.ds(r0, rpt)], acc_s.at[pl.ds(r0, rpt)])
        pltpu.sync_copy(zh.at[pl.ds(r0, rpt)], den_s.at[pl.ds(r0, rpt)])
        pltpu.sync_copy(zh.at[pl.ds(0, k)], exb.at[0])
        pltpu.sync_copy(zh.at[pl.ds(0, k)], exb.at[1])
        plsc.subcore_barrier()
        base0 = s * per_tile

        def issue(ci, b):
            base = base0 + ci * k
            pltpu.sync_copy(srcd.at[pl.ds(base, k)], src_v.at[b])
            pltpu.sync_copy(dstd.at[pl.ds(base, k)], dst_v.at[b])
            pltpu.async_copy(as_h.at[src_v.at[b]], asr.at[b], sem)
            pltpu.async_copy(ad_h.at[dst_v.at[b]], adr.at[b], sem)
            pltpu.async_copy(h_h.at[c].at[src_v.at[b]], hrows.at[b], sem)

        def wait_gathers(b):
            pltpu.make_async_copy(as_h.at[src_v.at[b]], asr.at[b], sem).wait()
            pltpu.make_async_copy(ad_h.at[dst_v.at[b]], adr.at[b], sem).wait()
            pltpu.make_async_copy(
                h_h.at[c].at[src_v.at[b]], hrows.at[b], sem).wait()

        def compute(b):
            def group(g, carry2):
                k16 = g * 16 + iota16
                for lh in range(H):
                    gcol = jnp.full((16,), c * H + lh, jnp.int32)
                    a1 = plsc.load_gather(asr.at[b], [k16, gcol])
                    a2 = plsc.load_gather(adr.at[b], [k16, gcol])
                    e = a1 + a2
                    ex = jnp.exp(jnp.where(e > 0, e, 0.2 * e))
                    plsc.store_scatter(
                        exb.at[b], [k16, jnp.full((16,), lh, jnp.int32)], ex)
                    for j in range(16):
                        exv = jnp.full((16,), ex[j], jnp.float32)
                        kk = g * 16 + j
                        for t in range(Cd // 16):
                            col = lh * Cd + t * 16
                            hv = hrows.at[b][kk, pl.ds(col, 16)]
                            hrows.at[b][kk, pl.ds(col, 16)] = exv * hv
                return carry2

            lax.fori_loop(0, groups, group, 0)

        issue(0, 0)

        def pair(i, carry):
            for b in (0, 1):
                wait_gathers(b)
                if b == 0:
                    issue(2 * i + 1, 1)
                else:
                    @pl.when(i < n_pairs - 1)
                    def _():
                        issue(2 * i + 2, 0)
                compute(b)
                pltpu.sync_copy(hrows.at[b], acc_s.at[dst_v.at[b]], add=True)
                pltpu.sync_copy(exb.at[b], den_s.at[dst_v.at[b]], add=True)
            return carry

        lax.fori_loop(0, n_pairs, pair, 0)
        plsc.subcore_barrier()
        pltpu.sync_copy(acc_s.at[pl.ds(r0, rpt)], acc_o.at[c].at[pl.ds(r0, rpt)])
        pltpu.sync_copy(den_s.at[pl.ds(r0, rpt)], den_o.at[c].at[pl.ds(r0, rpt)])

    return pl.kernel(
        body,
        out_type=(
            jax.ShapeDtypeStruct((2, npad, C), jnp.float32),
            jax.ShapeDtypeStruct((2, npad, 8), jnp.float32),
        ),
        mesh=mesh,
        compiler_params=pltpu.CompilerParams(needs_layout_passes=False, use_tc_tiling_on_sc=False),
        scratch_types=[
            pltpu.VMEM((2, k), jnp.int32),
            pltpu.VMEM((2, k), jnp.int32),
            pltpu.VMEM((2, k, 2 * H), jnp.float32),
            pltpu.VMEM((2, k, 2 * H), jnp.float32),
            pltpu.VMEM((2, k, C), jnp.float32),
            pltpu.VMEM((2, k, 8), jnp.float32),
            pltpu.VMEM_SHARED((npad, C), jnp.float32),
            pltpu.VMEM_SHARED((npad, 8), jnp.float32),
            pltpu.SemaphoreType.DMA,
        ],
    )


def _sweep3(epad, npad, k):
    mesh = plsc.VectorSubcoreMesh(core_axis_name="c", subcore_axis_name="s", num_cores=2, num_subcores=16)
    per_core = epad // 2
    per_tile = per_core // N_TILES
    n_chunks = per_tile // k
    rpt = npad // N_TILES
    groups = k // 16

    def body(srcd, dstd, as_h, ad_h, h_h, zc, acc_o,
             src_v, dst_v, asr, adr, hrows, contrib, acc_s, sem):
        c = lax.axis_index("c")
        s = lax.axis_index("s")
        iota16 = lax.iota(jnp.int32, 16)
        r0 = s * rpt
        pltpu.sync_copy(zc.at[pl.ds(r0, rpt)], acc_s.at[pl.ds(r0, rpt)])
        pltpu.sync_copy(zc.at[pl.ds(0, k)], contrib.at[0])
        pltpu.sync_copy(zc.at[pl.ds(0, k)], contrib.at[1])
        plsc.subcore_barrier()
        base0 = c * per_core + s * per_tile

        def issue(ci, b):
            base = base0 + ci * k
            pltpu.sync_copy(srcd.at[pl.ds(base, k)], src_v.at[b])
            pltpu.sync_copy(dstd.at[pl.ds(base, k)], dst_v.at[b])
            pltpu.async_copy(as_h.at[src_v.at[b]], asr.at[b], sem)
            pltpu.async_copy(ad_h.at[dst_v.at[b]], adr.at[b], sem)
            pltpu.async_copy(h_h.at[src_v.at[b]], hrows.at[b], sem)

        def wait_gathers(b):
            pltpu.make_async_copy(as_h.at[src_v.at[b]], asr.at[b], sem).wait()
            pltpu.make_async_copy(ad_h.at[dst_v.at[b]], adr.at[b], sem).wait()
            pltpu.make_async_copy(h_h.at[src_v.at[b]], hrows.at[b], sem).wait()

        def compute(b):
            def group(g, carry2):
                k16 = g * 16 + iota16
                a1 = plsc.load_gather(asr.at[b], [k16])
                a2 = plsc.load_gather(adr.at[b], [k16])
                e = a1 + a2
                ex = jnp.exp(jnp.where(e > 0, e, 0.2 * e))
                plsc.store_scatter(
                    contrib.at[b], [k16, jnp.full((16,), 16, jnp.int32)], ex)
                for j in range(16):
                    exv = jnp.full((16,), ex[j], jnp.float32)
                    kk = g * 16 + j
                    hv = hrows.at[b][kk, pl.ds(0, 16)]
                    contrib.at[b][kk, pl.ds(0, 16)] = exv * hv
                return carry2

            lax.fori_loop(0, groups, group, 0)

        n_pairs = n_chunks // 2
        issue(0, 0)

        def pair(i, carry):
            for b in (0, 1):
                wait_gathers(b)
                if b == 0:
                    issue(2 * i + 1, 1)
                else:
                    @pl.when(i < n_pairs - 1)
                    def _():
                        issue(2 * i + 2, 0)
                compute(b)
                pltpu.sync_copy(contrib.at[b], acc_s.at[dst_v.at[b]], add=True)
            return carry

        lax.fori_loop(0, n_pairs, pair, 0)
        plsc.subcore_barrier()
        pltpu.sync_copy(acc_s.at[pl.ds(r0, rpt)], acc_o.at[c].at[pl.ds(r0, rpt)])

    return pl.kernel(
        body,
        out_type=jax.ShapeDtypeStruct((2, npad, 32), jnp.float32),
        mesh=mesh,
        compiler_params=pltpu.CompilerParams(needs_layout_passes=False, use_tc_tiling_on_sc=False),
        scratch_types=[
            pltpu.VMEM((2, k), jnp.int32),
            pltpu.VMEM((2, k), jnp.int32),
            pltpu.VMEM((2, k), jnp.float32),
            pltpu.VMEM((2, k), jnp.float32),
            pltpu.VMEM((2, k, 16), jnp.float32),
            pltpu.VMEM((2, k, 32), jnp.float32),
            pltpu.VMEM_SHARED((npad, 32), jnp.float32),
            pltpu.SemaphoreType.DMA,
        ],
    )


def _tc_first(x, W, As, Ad, C):
    n = x.shape[0]
    heads = As.shape[1]

    def body(x_ref, w_ref, as_ref, ad_ref, h_out, s_out, d_out):
        h = jnp.dot(x_ref[...], w_ref[...], preferred_element_type=jnp.float32)
        s_out[...] = jnp.dot(h, as_ref[...], preferred_element_type=jnp.float32)
        d_out[...] = jnp.dot(h, ad_ref[...], preferred_element_type=jnp.float32)
        h_out[0] = h[:, :C]
        h_out[1] = h[:, C:]

    return pl.pallas_call(
        body,
        out_shape=(
            jax.ShapeDtypeStruct((2, n, C), jnp.float32),
            jax.ShapeDtypeStruct((n, heads), jnp.float32),
            jax.ShapeDtypeStruct((n, heads), jnp.float32),
        ),
    )(x, W, As, Ad)


def _tc_norm(acc, den, b, g, be, Cd, H):
    n = acc.shape[1]

    def body(acc_ref, den_ref, b_ref, g_ref, be_ref, x_out):
        parts = []
        for c in range(2):
            a = acc_ref[c]
            d = den_ref[c]
            denr = jnp.concatenate(
                [jnp.broadcast_to(d[:, h:h + 1], (n, Cd)) for h in range(H)],
                axis=1)
            parts.append(a / (denr + 1e-16))
        o = jnp.concatenate(parts, axis=1) + b_ref[...]
        mu = jnp.mean(o, axis=0)
        var = jnp.mean((o - mu) ** 2, axis=0)
        xb = (o - mu) * lax.rsqrt(var + 1e-5) * g_ref[...] + be_ref[...]
        x_out[...] = jnp.where(xb > 0, xb, jnp.exp(xb) - 1.0)

    return pl.pallas_call(
        body,
        out_shape=jax.ShapeDtypeStruct((n, 2 * acc.shape[2]), jnp.float32),
        compiler_params=pltpu.CompilerParams(
            vmem_limit_bytes=64 * 1024 * 1024),
    )(acc, den, b, g, be)


def _tc_mid(acc, den, b, g, be, W, As, Ad, Cd, H, split_out):
    xa = _tc_norm(acc, den, b, g, be, Cd, H)
    n = xa.shape[0]
    Fout = W.shape[1]
    heads = As.shape[1]

    def body(x_ref, w_ref, as_ref, ad_ref, h_out, s_out, d_out):
        h = jnp.dot(x_ref[...], w_ref[...], preferred_element_type=jnp.float32)
        s_out[...] = jnp.dot(h, as_ref[...], preferred_element_type=jnp.float32)
        d_out[...] = jnp.dot(h, ad_ref[...], preferred_element_type=jnp.float32)
        if split_out:
            h_out[0] = h[:, :Fout // 2]
            h_out[1] = h[:, Fout // 2:]
        else:
            h_out[...] = h

    h_shape = ((2, n, Fout // 2) if split_out else (n, Fout))
    return pl.pallas_call(
        body,
        out_shape=(
            jax.ShapeDtypeStruct(h_shape, jnp.float32),
            jax.ShapeDtypeStruct((n, heads), jnp.float32),
            jax.ShapeDtypeStruct((n, heads), jnp.float32),
        ),
    )(xa, W, As, Ad)


def _tc_final(acc, b):
    n = acc.shape[1]

    def body(acc_ref, b_ref, out_ref):
        a = acc_ref[0] + acc_ref[1]
        h = a[:, :16] / (a[:, 16:17] + 1e-16) + b_ref[...]
        m = jnp.max(h, axis=1, keepdims=True)
        lse = m + jnp.log(jnp.sum(jnp.exp(h - m), axis=1, keepdims=True))
        out_ref[...] = h - lse

    return pl.pallas_call(
        body,
        out_shape=jax.ShapeDtypeStruct((n, 16), jnp.float32),
    )(acc, b)


def _block_diag_proj(a):
    heads, Cd = a.shape
    A = a[:, :, None] * jnp.eye(heads, dtype=a.dtype)[:, None, :]
    return A.reshape(heads * Cd, heads)


def _pad_rows(a, npad):
    pad = [(0, npad - a.shape[0])] + [(0, 0)] * (a.ndim - 1)
    return jnp.pad(a, pad)


def kernel(x, edge_index, W1, a1s, a1d, b1, g1, be1,
           W2, a2s, a2d, b2, g2, be2, W3, a3s, a3d, b3):
    n, _ = x.shape
    e = edge_index.shape[1]
    e_tot = e + n
    epad = _round_up(e_tot, 64 * K_CH)
    npad = NPAD

    loops = jnp.arange(n, dtype=jnp.int32)
    srcd = jnp.concatenate(
        [edge_index[0].astype(jnp.int32), loops,
         jnp.zeros((epad - e_tot,), jnp.int32)])
    dstd = jnp.concatenate(
        [edge_index[1].astype(jnp.int32), loops,
         jnp.full((epad - e_tot,), n, jnp.int32)])

    A1s, A1d = _block_diag_proj(a1s), _block_diag_proj(a1d)
    A2s, A2d = _block_diag_proj(a2s), _block_diag_proj(a2d)
    A3s, A3d = _block_diag_proj(a3s), _block_diag_proj(a3d)

    zc64 = jnp.zeros((npad, 64), jnp.float32)
    zc128 = jnp.zeros((npad, 128), jnp.float32)
    zc32 = jnp.zeros((npad, 32), jnp.float32)
    zh8 = jnp.zeros((npad, 8), jnp.float32)

    h1, as1, ad1 = _tc_first(x, W1, A1s, A1d, 64)
    acc1, den1 = _sweep12(64, 16, 4, epad, npad, K_CH)(
        srcd, dstd, _pad_rows(as1, npad), _pad_rows(ad1, npad),
        h1, zc64, zh8)

    h2, as2, ad2 = _tc_mid(acc1[:, :n], den1[:, :n], b1, g1, be1,
                           W2, A2s, A2d, 16, 4, True)
    acc2, den2 = _sweep12(128, 32, 4, epad, npad, K_CH)(
        srcd, dstd, _pad_rows(as2, npad), _pad_rows(ad2, npad),
        h2, zc128, zh8)

    h3, as3, ad3 = _tc_mid(acc2[:, :n], den2[:, :n], b2, g2, be2,
                           W3, A3s, A3d, 32, 4, False)
    acc3 = _sweep3(epad, npad, K_CH)(
        srcd, dstd, _pad_rows(as3[:, 0], npad), _pad_rows(ad3[:, 0], npad),
        h3, zc32)

    return _tc_final(acc3[:, :n], b3)


# speedup vs baseline: 3.6202x; 1.0008x over previous
import jax
import jax.numpy as jnp
from jax import lax
from jax.experimental import pallas as pl
from jax.experimental.pallas import tpu as pltpu
import jax.experimental.pallas.tpu_sc as plsc

NPAD = 10112
N_TILES = 16
K_CH = 128


def _round_up(v, m):
    return ((v + m - 1) // m) * m


def _sweep12(C, Cd, H, epad, npad, k):
    mesh = plsc.VectorSubcoreMesh(core_axis_name="c", subcore_axis_name="s", num_cores=2, num_subcores=16)
    per_tile = epad // N_TILES
    n_chunks = per_tile // k
    rpt = npad // N_TILES
    groups = k // 16

    n_pairs = n_chunks // 2

    def body(srcd, dstd, as_h, ad_h, h_h, zc, zh, acc_o, den_o,
             src_v, dst_v, asr, adr, hrows, exb, acc_s, den_s, sem):
        c = lax.axis_index("c")
        s = lax.axis_index("s")
        iota16 = lax.iota(jnp.int32, 16)
        r0 = s * rpt
        pltpu.sync_copy(zc.at[pl.ds(r0, rpt)], acc_s.at[pl.ds(r0, rpt)])
        pltpu.sync_copy(zh.at[pl.ds(r0, rpt)], den_s.at[pl.ds(r0, rpt)])
        pltpu.sync_copy(zh.at[pl.ds(0, k)], exb.at[0])
        pltpu.sync_copy(zh.at[pl.ds(0, k)], exb.at[1])
        plsc.subcore_barrier()
        base0 = s * per_tile

        def issue(ci, b):
            base = base0 + ci * k
            pltpu.sync_copy(srcd.at[pl.ds(base, k)], src_v.at[b])
            pltpu.sync_copy(dstd.at[pl.ds(base, k)], dst_v.at[b])
            pltpu.async_copy(as_h.at[src_v.at[b]], asr.at[b], sem)
            pltpu.async_copy(ad_h.at[dst_v.at[b]], adr.at[b], sem)
            pltpu.async_copy(h_h.at[c].at[src_v.at[b]], hrows.at[b], sem)

        def wait_gathers(b):
            pltpu.make_async_copy(as_h.at[src_v.at[b]], asr.at[b], sem).wait()
            pltpu.make_async_copy(ad_h.at[dst_v.at[b]], adr.at[b], sem).wait()
            pltpu.make_async_copy(
                h_h.at[c].at[src_v.at[b]], hrows.at[b], sem).wait()

        def compute(b):
            def group(g, carry2):
                k16 = g * 16 + iota16
                for lh in range(H):
                    gcol = jnp.full((16,), c * H + lh, jnp.int32)
                    a1 = plsc.load_gather(asr.at[b], [k16, gcol])
                    a2 = plsc.load_gather(adr.at[b], [k16, gcol])
                    e = a1 + a2
                    ex = jnp.exp(jnp.where(e > 0, e, 0.2 * e))
                    plsc.store_scatter(
                        exb.at[b], [k16, jnp.full((16,), lh, jnp.int32)], ex)
                    for j in range(16):
                        exv = jnp.full((16,), ex[j], jnp.float32)
                        kk = g * 16 + j
                        for t in range(Cd // 16):
                            col = lh * Cd + t * 16
                            hv = hrows.at[b][kk, pl.ds(col, 16)]
                            hrows.at[b][kk, pl.ds(col, 16)] = exv * hv
                return carry2

            lax.fori_loop(0, groups, group, 0)

        issue(0, 0)

        def pair(i, carry):
            for b in (0, 1):
                wait_gathers(b)
                if b == 0:
                    issue(2 * i + 1, 1)
                else:
                    @pl.when(i < n_pairs - 1)
                    def _():
                        issue(2 * i + 2, 0)
                compute(b)
                pltpu.sync_copy(hrows.at[b], acc_s.at[dst_v.at[b]], add=True)
                pltpu.sync_copy(exb.at[b], den_s.at[dst_v.at[b]], add=True)
            return carry

        lax.fori_loop(0, n_pairs, pair, 0)
        plsc.subcore_barrier()
        pltpu.sync_copy(acc_s.at[pl.ds(r0, rpt)], acc_o.at[c].at[pl.ds(r0, rpt)])
        pltpu.sync_copy(den_s.at[pl.ds(r0, rpt)], den_o.at[c].at[pl.ds(r0, rpt)])

    return pl.kernel(
        body,
        out_type=(
            jax.ShapeDtypeStruct((2, npad, C), jnp.float32),
            jax.ShapeDtypeStruct((2, npad, 8), jnp.float32),
        ),
        mesh=mesh,
        compiler_params=pltpu.CompilerParams(needs_layout_passes=False, use_tc_tiling_on_sc=False),
        scratch_types=[
            pltpu.VMEM((2, k), jnp.int32),
            pltpu.VMEM((2, k), jnp.int32),
            pltpu.VMEM((2, k, 2 * H), jnp.float32),
            pltpu.VMEM((2, k, 2 * H), jnp.float32),
            pltpu.VMEM((2, k, C), jnp.float32),
            pltpu.VMEM((2, k, 8), jnp.float32),
            pltpu.VMEM_SHARED((npad, C), jnp.float32),
            pltpu.VMEM_SHARED((npad, 8), jnp.float32),
            pltpu.SemaphoreType.DMA,
        ],
    )


def _sweep3(epad, npad, k):
    mesh = plsc.VectorSubcoreMesh(core_axis_name="c", subcore_axis_name="s", num_cores=2, num_subcores=16)
    per_core = epad // 2
    per_tile = per_core // N_TILES
    n_chunks = per_tile // k
    rpt = npad // N_TILES
    groups = k // 16

    def body(srcd, dstd, as_h, ad_h, h_h, zc, acc_o,
             src_v, dst_v, asr, adr, hrows, contrib, acc_s, sem):
        c = lax.axis_index("c")
        s = lax.axis_index("s")
        iota16 = lax.iota(jnp.int32, 16)
        r0 = s * rpt
        pltpu.sync_copy(zc.at[pl.ds(r0, rpt)], acc_s.at[pl.ds(r0, rpt)])
        pltpu.sync_copy(zc.at[pl.ds(0, k)], contrib.at[0])
        pltpu.sync_copy(zc.at[pl.ds(0, k)], contrib.at[1])
        plsc.subcore_barrier()
        base0 = c * per_core + s * per_tile

        def issue(ci, b):
            base = base0 + ci * k
            pltpu.sync_copy(srcd.at[pl.ds(base, k)], src_v.at[b])
            pltpu.sync_copy(dstd.at[pl.ds(base, k)], dst_v.at[b])
            pltpu.async_copy(as_h.at[src_v.at[b]], asr.at[b], sem)
            pltpu.async_copy(ad_h.at[dst_v.at[b]], adr.at[b], sem)
            pltpu.async_copy(h_h.at[src_v.at[b]], hrows.at[b], sem)

        def wait_gathers(b):
            pltpu.make_async_copy(as_h.at[src_v.at[b]], asr.at[b], sem).wait()
            pltpu.make_async_copy(ad_h.at[dst_v.at[b]], adr.at[b], sem).wait()
            pltpu.make_async_copy(h_h.at[src_v.at[b]], hrows.at[b], sem).wait()

        def compute(b):
            def group(g, carry2):
                k16 = g * 16 + iota16
                a1 = plsc.load_gather(asr.at[b], [k16])
                a2 = plsc.load_gather(adr.at[b], [k16])
                e = a1 + a2
                ex = jnp.exp(jnp.where(e > 0, e, 0.2 * e))
                plsc.store_scatter(
                    contrib.at[b], [k16, jnp.full((16,), 16, jnp.int32)], ex)
                for j in range(16):
                    exv = jnp.full((16,), ex[j], jnp.float32)
                    kk = g * 16 + j
                    hv = hrows.at[b][kk, pl.ds(0, 16)]
                    contrib.at[b][kk, pl.ds(0, 16)] = exv * hv
                return carry2

            lax.fori_loop(0, groups, group, 0)

        n_pairs = n_chunks // 2
        issue(0, 0)

        def pair(i, carry):
            for b in (0, 1):
                wait_gathers(b)
                if b == 0:
                    issue(2 * i + 1, 1)
                else:
                    @pl.when(i < n_pairs - 1)
                    def _():
                        issue(2 * i + 2, 0)
                compute(b)
                pltpu.sync_copy(contrib.at[b], acc_s.at[dst_v.at[b]], add=True)
            return carry

        lax.fori_loop(0, n_pairs, pair, 0)
        plsc.subcore_barrier()
        pltpu.sync_copy(acc_s.at[pl.ds(r0, rpt)], acc_o.at[c].at[pl.ds(r0, rpt)])

    return pl.kernel(
        body,
        out_type=jax.ShapeDtypeStruct((2, npad, 32), jnp.float32),
        mesh=mesh,
        compiler_params=pltpu.CompilerParams(needs_layout_passes=False, use_tc_tiling_on_sc=False),
        scratch_types=[
            pltpu.VMEM((2, k), jnp.int32),
            pltpu.VMEM((2, k), jnp.int32),
            pltpu.VMEM((2, k), jnp.float32),
            pltpu.VMEM((2, k), jnp.float32),
            pltpu.VMEM((2, k, 16), jnp.float32),
            pltpu.VMEM((2, k, 32), jnp.float32),
            pltpu.VMEM_SHARED((npad, 32), jnp.float32),
            pltpu.SemaphoreType.DMA,
        ],
    )


def _tc_first(x, W, As, Ad, C):
    n = x.shape[0]
    heads = As.shape[1]

    def body(x_ref, w_ref, as_ref, ad_ref, h_out, s_out, d_out):
        h = jnp.dot(x_ref[...], w_ref[...], preferred_element_type=jnp.float32)
        s_out[...] = jnp.dot(h, as_ref[...], preferred_element_type=jnp.float32)
        d_out[...] = jnp.dot(h, ad_ref[...], preferred_element_type=jnp.float32)
        h_out[0] = h[:, :C]
        h_out[1] = h[:, C:]

    return pl.pallas_call(
        body,
        out_shape=(
            jax.ShapeDtypeStruct((2, n, C), jnp.float32),
            jax.ShapeDtypeStruct((n, heads), jnp.float32),
            jax.ShapeDtypeStruct((n, heads), jnp.float32),
        ),
    )(x, W, As, Ad)


def _tc_norm(acc, den, b, g, be, Cd, H):
    n = acc.shape[1]

    def body(acc_ref, den_ref, b_ref, g_ref, be_ref, x_out):
        parts = []
        for c in range(2):
            a = acc_ref[c]
            d = den_ref[c]
            denr = jnp.concatenate(
                [jnp.broadcast_to(d[:, h:h + 1], (n, Cd)) for h in range(H)],
                axis=1)
            parts.append(a / (denr + 1e-16))
        o = jnp.concatenate(parts, axis=1) + b_ref[...]
        mu = jnp.mean(o, axis=0)
        var = jnp.mean((o - mu) ** 2, axis=0)
        xb = (o - mu) * lax.rsqrt(var + 1e-5) * g_ref[...] + be_ref[...]
        x_out[...] = jnp.where(xb > 0, xb, jnp.exp(xb) - 1.0)

    return pl.pallas_call(
        body,
        out_shape=jax.ShapeDtypeStruct((n, 2 * acc.shape[2]), jnp.float32),
        compiler_params=pltpu.CompilerParams(
            vmem_limit_bytes=64 * 1024 * 1024),
    )(acc, den, b, g, be)


def _tc_mid(acc, den, b, g, be, W, As, Ad, Cd, H, split_out):
    xa = _tc_norm(acc, den, b, g, be, Cd, H)
    n = xa.shape[0]
    Fout = W.shape[1]
    heads = As.shape[1]

    def body(x_ref, w_ref, as_ref, ad_ref, h_out, s_out, d_out):
        h = jnp.dot(x_ref[...], w_ref[...], preferred_element_type=jnp.float32)
        s_out[...] = jnp.dot(h, as_ref[...], preferred_element_type=jnp.float32)
        d_out[...] = jnp.dot(h, ad_ref[...], preferred_element_type=jnp.float32)
        if split_out:
            h_out[0] = h[:, :Fout // 2]
            h_out[1] = h[:, Fout // 2:]
        else:
            h_out[...] = h

    h_shape = ((2, n, Fout // 2) if split_out else (n, Fout))
    return pl.pallas_call(
        body,
        out_shape=(
            jax.ShapeDtypeStruct(h_shape, jnp.float32),
            jax.ShapeDtypeStruct((n, heads), jnp.float32),
            jax.ShapeDtypeStruct((n, heads), jnp.float32),
        ),
    )(xa, W, As, Ad)


def _tc_final(acc, b):
    n = acc.shape[1]

    def body(acc_ref, b_ref, out_ref):
        a = acc_ref[0] + acc_ref[1]
        h = a[:, :16] / (a[:, 16:17] + 1e-16) + b_ref[...]
        m = jnp.max(h, axis=1, keepdims=True)
        lse = m + jnp.log(jnp.sum(jnp.exp(h - m), axis=1, keepdims=True))
        out_ref[...] = h - lse

    return pl.pallas_call(
        body,
        out_shape=jax.ShapeDtypeStruct((n, 16), jnp.float32),
    )(acc, b)


def _block_diag_proj(a):
    heads, Cd = a.shape
    A = a[:, :, None] * jnp.eye(heads, dtype=a.dtype)[:, None, :]
    return A.reshape(heads * Cd, heads)


def _pad_rows(a, npad):
    pad = [(0, npad - a.shape[0])] + [(0, 0)] * (a.ndim - 1)
    return jnp.pad(a, pad)


def kernel(x, edge_index, W1, a1s, a1d, b1, g1, be1,
           W2, a2s, a2d, b2, g2, be2, W3, a3s, a3d, b3):
    n, _ = x.shape
    e = edge_index.shape[1]
    e_tot = e + n
    epad = _round_up(e_tot, 64 * K_CH)
    npad = NPAD

    loops = jnp.arange(n, dtype=jnp.int32)
    srcd = jnp.concatenate(
        [edge_index[0].astype(jnp.int32), loops,
         jnp.zeros((epad - e_tot,), jnp.int32)])
    dstd = jnp.concatenate(
        [edge_index[1].astype(jnp.int32), loops,
         jnp.full((epad - e_tot,), n, jnp.int32)])

    A1s, A1d = _block_diag_proj(a1s), _block_diag_proj(a1d)
    A2s, A2d = _block_diag_proj(a2s), _block_diag_proj(a2d)
    A3s, A3d = _block_diag_proj(a3s), _block_diag_proj(a3d)

    zc64 = jnp.zeros((npad, 64), jnp.float32)
    zc128 = jnp.zeros((npad, 128), jnp.float32)
    zc32 = jnp.zeros((npad, 32), jnp.float32)
    zh8 = jnp.zeros((npad, 8), jnp.float32)

    h1, as1, ad1 = _tc_first(x, W1, A1s, A1d, 64)
    acc1, den1 = _sweep12(64, 16, 4, epad, npad, K_CH)(
        srcd, dstd, _pad_rows(as1, npad), _pad_rows(ad1, npad),
        h1, zc64, zh8)

    h2, as2, ad2 = _tc_mid(acc1[:, :n], den1[:, :n], b1, g1, be1,
                           W2, A2s, A2d, 16, 4, True)
    acc2, den2 = _sweep12(128, 32, 4, epad, npad, K_CH)(
        srcd, dstd, _pad_rows(as2, npad), _pad_rows(ad2, npad),
        h2, zc128, zh8)

    h3, as3, ad3 = _tc_mid(acc2[:, :n], den2[:, :n], b2, g2, be2,
                           W3, A3s, A3d, 32, 4, False)
    acc3 = _sweep3(epad, npad, K_CH)(
        srcd, dstd, _pad_rows(as3[:, 0], npad), _pad_rows(ad3[:, 0], npad),
        h3, zc32)

    return _tc_final(acc3[:, :n], b3)
